# Initial kernel scaffold; baseline (speedup 1.0000x reference)
import jax
import jax.numpy as jnp
from jax.experimental import pallas as pl


def kernel(node_reps, edges, edge_weights, prep_gamma, prep_beta, prep_mean, prep_var, prep_W, prep_b, upd_gamma, upd_beta, upd_mean, upd_var, upd_W, upd_b):
    raise NotImplementedError("write your pallas kernel here")


# speedup vs baseline: 2.7742x; 2.7742x over previous
import functools

import jax
import jax.numpy as jnp
from jax import lax
from jax.experimental import pallas as pl
from jax.experimental.pallas import tpu as pltpu
from jax.experimental.pallas import tpu_sc as plsc

N = 10000
E = 160000
D = 256
H = 128
NC = 2
NS = 16
ET = E // NS
K = 80
G = ET // K
RT = 640
RTL = N - (NS - 1) * RT
E2 = E // NC
ETC = E2 // NS
KC = 40
GC = ETC // KC

f32 = jnp.float32
i32 = jnp.int32


def _prep_body(x_ref, w_ref, b_ref, o_ref):
    y = jnp.dot(x_ref[...], w_ref[...], preferred_element_type=f32)
    o_ref[...] = jnp.maximum(y + b_ref[...], 0.0)


def _prep_call(x, w1, b1):
    rb = 1000
    grid = (N // rb, 2)
    return pl.pallas_call(
        _prep_body,
        grid=grid,
        in_specs=[
            pl.BlockSpec((rb, D), lambda i, h: (i, 0)),
            pl.BlockSpec((D, H), lambda i, h: (0, h)),
            pl.BlockSpec((1, H), lambda i, h: (0, h)),
        ],
        out_specs=pl.BlockSpec((rb, H), lambda i, h: (h * (N // rb) + i, 0)),
        out_shape=jax.ShapeDtypeStruct((2 * N, H), f32),
    )(x, w1, b1)


def _sc_body(m_hbm, srcoff_hbm, dst_hbm, w_hbm, sums_hbm, cnt_hbm,
         acc, idx_s, idx_d, idx_c, wv, rows, sem):
    c = lax.axis_index("c")
    s = lax.axis_index("s")

    def split_copy(mk):
        @pl.when(s < NS - 1)
        def _():
            for k2 in range(RT // K):
                mk(s * RT + k2 * K)

        @pl.when(s == NS - 1)
        def _():
            for k2 in range(RTL // K):
                mk((NS - 1) * RT + k2 * K)

    def fill(val):
        v16 = jnp.full((16,), val, f32)

        def body_(r, carry2):
            for j in range(H // 16):
                rows[r, pl.ds(j * 16, 16)] = v16
            return carry2

        lax.fori_loop(0, K, body_, 0)

    fill(0.0)
    split_copy(lambda b: pltpu.sync_copy(rows, acc.at[pl.ds(b, K)]))
    plsc.subcore_barrier()

    ebase = s * ET

    def chunk(g, carry):
        base = ebase + g * K
        pltpu.sync_copy(srcoff_hbm.at[pl.ds(c * E + base, K)], idx_s)
        pltpu.sync_copy(dst_hbm.at[pl.ds(base, K)], idx_d)
        pltpu.sync_copy(w_hbm.at[pl.ds(base, K)], wv)
        pltpu.async_copy(m_hbm.at[idx_s], rows, sem).wait()

        def rgroup(r, carry2):
            w16 = wv[pl.ds(r * 16, 16)]
            for e in range(16):
                row = r * 16 + e
                wb = jnp.broadcast_to(w16[e], (16,))
                for j in range(H // 16):
                    sl = pl.ds(j * 16, 16)
                    rows[row, sl] = rows[row, sl] * wb
            return carry2

        lax.fori_loop(0, K // 16, rgroup, 0)
        pltpu.sync_copy(rows, acc.at[idx_d], add=True)
        return carry

    lax.fori_loop(0, G, chunk, 0)
    plsc.subcore_barrier()

    def wb_row(b):
        pltpu.sync_copy(acc.at[pl.ds(b, K)], rows)
        pltpu.sync_copy(rows, sums_hbm.at[c, pl.ds(b, K)])

    split_copy(wb_row)
    plsc.subcore_barrier()

    fill(0.0)
    split_copy(lambda b: pltpu.sync_copy(rows, acc.at[pl.ds(b, K)]))
    fill(1.0)
    plsc.subcore_barrier()

    cbase = c * E2 + s * ETC

    def cchunk(g, carry):
        pltpu.sync_copy(dst_hbm.at[pl.ds(cbase + g * KC, KC)], idx_c)
        pltpu.sync_copy(rows.at[pl.ds(0, KC)], acc.at[idx_c], add=True)
        return carry

    lax.fori_loop(0, GC, cchunk, 0)
    plsc.subcore_barrier()

    def wb_cnt(b):
        pltpu.sync_copy(acc.at[pl.ds(b, K)], rows)
        pltpu.sync_copy(rows, cnt_hbm.at[c, pl.ds(b, K)])

    split_copy(wb_cnt)


def _sc_call(m, srcoff, dst, w):
    mesh = plsc.VectorSubcoreMesh(core_axis_name="c", subcore_axis_name="s")
    kern = pl.kernel(
        _sc_body,
        out_type=(jax.ShapeDtypeStruct((NC, N, H), f32),
                  jax.ShapeDtypeStruct((NC, N, H), f32)),
        mesh=mesh,
        scratch_types=(
            pltpu.VMEM_SHARED((N, H), f32),
            pltpu.VMEM((K,), i32),
            pltpu.VMEM((K,), i32),
            pltpu.VMEM((KC,), i32),
            pltpu.VMEM((K,), f32),
            pltpu.VMEM((K, H), f32),
            pltpu.SemaphoreType.DMA,
        ),
    )
    return kern(m, srcoff, dst, w)


def _upd_body(x_ref, a0_ref, a1_ref, c0_ref, c1_ref, wa_ref, w0_ref, w1_ref,
              b_ref, o_ref):
    cnt = c0_ref[...][:, 0:1] + c1_ref[...][:, 0:1]
    inv = 1.0 / jnp.maximum(cnt, 1.0)
    y = jnp.dot(x_ref[...], wa_ref[...], preferred_element_type=f32)
    y += jnp.dot(a0_ref[...] * inv, w0_ref[...], preferred_element_type=f32)
    y += jnp.dot(a1_ref[...] * inv, w1_ref[...], preferred_element_type=f32)
    o_ref[...] = jnp.maximum(y + b_ref[...], 0.0)


def _upd_call(x, a0, a1, c0, c1, wa, w0, w1, b2):
    rb = 1000
    grid = (N // rb,)
    return pl.pallas_call(
        _upd_body,
        grid=grid,
        in_specs=[
            pl.BlockSpec((rb, D), lambda i: (i, 0)),
            pl.BlockSpec((rb, H), lambda i: (i, 0)),
            pl.BlockSpec((rb, H), lambda i: (i, 0)),
            pl.BlockSpec((rb, H), lambda i: (i, 0)),
            pl.BlockSpec((rb, H), lambda i: (i, 0)),
            pl.BlockSpec((D, D), lambda i: (0, 0)),
            pl.BlockSpec((H, D), lambda i: (0, 0)),
            pl.BlockSpec((H, D), lambda i: (0, 0)),
            pl.BlockSpec((1, D), lambda i: (0, 0)),
        ],
        out_specs=pl.BlockSpec((rb, D), lambda i: (i, 0)),
        out_shape=jax.ShapeDtypeStruct((N, D), f32),
    )(x, a0, a1, c0, c1, wa, w0, w1, b2)


@jax.jit
def _run(node_reps, edges, edge_weights, prep_gamma, prep_beta, prep_mean,
         prep_var, prep_W, prep_b, upd_gamma, upd_beta, upd_mean, upd_var,
         upd_W, upd_b):
    ap = prep_gamma / jnp.sqrt(prep_var + 1e-3)
    cp = prep_beta - prep_mean * ap
    w1 = prep_W * ap[:, None]
    b1 = (cp @ prep_W + prep_b)[None, :]

    au = upd_gamma / jnp.sqrt(upd_var + 1e-3)
    cu = upd_beta - upd_mean * au
    w2 = upd_W * au[:, None]
    b2 = (cu @ upd_W + upd_b)[None, :]
    wa, w0, w1u = w2[:D], w2[D:D + H], w2[D + H:]

    m = _prep_call(node_reps, w1, b1)

    src = edges[1]
    srcoff = jnp.concatenate([src, src + N])
    dst = edges[0]
    sums, cnt = _sc_call(m, srcoff, dst, edge_weights)

    return _upd_call(node_reps, sums[0], sums[1], cnt[0], cnt[1],
                     wa, w0, w1u, b2)


def kernel(node_reps, edges, edge_weights, prep_gamma, prep_beta, prep_mean,
           prep_var, prep_W, prep_b, upd_gamma, upd_beta, upd_mean, upd_var,
           upd_W, upd_b):
    return _run(node_reps, edges, edge_weights, prep_gamma, prep_beta,
                prep_mean, prep_var, prep_W, prep_b, upd_gamma, upd_beta,
                upd_mean, upd_var, upd_W, upd_b)


# speedup vs baseline: 4.0634x; 1.4647x over previous
import functools

import jax
import jax.numpy as jnp
from jax import lax
from jax.experimental import pallas as pl
from jax.experimental.pallas import tpu as pltpu
from jax.experimental.pallas import tpu_sc as plsc

N = 10000
E = 160000
D = 256
H = 128
NC = 2
NS = 16
ET = E // NS
K = 80
G = ET // K
RT = 640
RTL = N - (NS - 1) * RT
E2 = E // NC
ETC = E2 // NS
KM = 128
ETP = 10240
GM = ETP // KM
ETCP = 5120
GCM = ETCP // KM

f32 = jnp.float32
i32 = jnp.int32


def _prep_body(x_ref, w_ref, b_ref, o_ref):
    y = jnp.dot(x_ref[...], w_ref[...], preferred_element_type=f32)
    o_ref[...] = jnp.maximum(y + b_ref[...], 0.0)


def _prep_call(x, w1, b1):
    rb = 1000
    grid = (N // rb, 2)
    return pl.pallas_call(
        _prep_body,
        grid=grid,
        in_specs=[
            pl.BlockSpec((rb, D), lambda i, h: (i, 0)),
            pl.BlockSpec((D, H), lambda i, h: (0, h)),
            pl.BlockSpec((1, H), lambda i, h: (0, h)),
        ],
        out_specs=pl.BlockSpec((rb, H), lambda i, h: (h * (N // rb) + i, 0)),
        out_shape=jax.ShapeDtypeStruct((2 * N, H), f32),
    )(x, w1, b1)


def _sc_body(m_hbm, src_hbm, w_hbm, dstm_hbm, cdstm_hbm, sums_hbm, cnt_hbm,
             acc, dstv, cdstv, b0, b1, sq0, sq1, wq0, wq1,
             sem_g, sem_i0, sem_i1, sem_c):
    c = lax.axis_index("c")
    s = lax.axis_index("s")

    def split_copy(mk):
        @pl.when(s < NS - 1)
        def _():
            for k2 in range(RT // K):
                mk(s * RT + k2 * K)

        @pl.when(s == NS - 1)
        def _():
            for k2 in range(RTL // K):
                mk((NS - 1) * RT + k2 * K)

    def fill(buf, val, nrows):
        v16 = jnp.full((16,), val, f32)

        def body_(r, carry2):
            for j in range(H // 16):
                buf[r, pl.ds(j * 16, 16)] = v16
            return carry2

        lax.fori_loop(0, nrows, body_, 0)

    pltpu.sync_copy(dstm_hbm.at[s], dstv)
    pltpu.sync_copy(cdstm_hbm.at[c * NS + s], cdstv)

    sbase = (c * NS + s) * ETP
    wbase = s * ETP

    def iissue(g, sq, wq, sem):
        pltpu.async_copy(src_hbm.at[pl.ds(sbase + g * KM, KM)], sq, sem)
        pltpu.async_copy(w_hbm.at[pl.ds(wbase + g * KM, KM)], wq, sem)

    def iwait(g, sq, wq, sem):
        pltpu.make_async_copy(src_hbm.at[pl.ds(sbase + g * KM, KM)], sq,
                              sem).wait()
        pltpu.make_async_copy(w_hbm.at[pl.ds(wbase + g * KM, KM)], wq,
                              sem).wait()

    def gissue(g, buf, sq):
        pltpu.async_copy(m_hbm.at[sq], buf, sem_g)

    def gwait(g, buf, sq):
        pltpu.make_async_copy(m_hbm.at[sq], buf, sem_g).wait()

    def mul(g, buf, wq):
        def rgroup(r, carry2):
            w16 = wq[pl.ds(r * 16, 16)]
            for e in range(16):
                row = r * 16 + e
                wb = jnp.broadcast_to(w16[e], (16,))
                for j in range(H // 16):
                    sl = pl.ds(j * 16, 16)
                    buf[row, sl] = buf[row, sl] * wb
            return carry2

        lax.fori_loop(0, KM // 16, rgroup, 0)

    fill(b0, 0.0, K)
    zsrc = b0.at[pl.ds(0, K)]
    split_copy(lambda b: pltpu.sync_copy(zsrc, acc.at[pl.ds(b, K)]))

    pltpu.sync_copy(src_hbm.at[pl.ds(sbase, KM)], sq0)
    pltpu.sync_copy(w_hbm.at[pl.ds(wbase, KM)], wq0)
    gissue(0, b0, sq0)
    iissue(1, sq1, wq1, sem_i1)
    plsc.subcore_barrier()

    def piter(jj, carry):
        ga = 2 * jj
        gb = 2 * jj + 1
        iwait(gb, sq1, wq1, sem_i1)
        gwait(ga, b0, sq0)
        gissue(gb, b1, sq1)
        mul(ga, b0, wq0)

        @pl.when(jj <= (GM // 2) - 2)
        def _():
            iissue(ga + 2, sq0, wq0, sem_i0)

        pltpu.sync_copy(b0, acc.at[dstv.at[ga]], add=True)
        gwait(gb, b1, sq1)

        @pl.when(jj <= (GM // 2) - 2)
        def _():
            iwait(ga + 2, sq0, wq0, sem_i0)
            gissue(gb + 1, b0, sq0)

        mul(gb, b1, wq1)

        @pl.when(jj <= (GM // 2) - 2)
        def _():
            iissue(gb + 2, sq1, wq1, sem_i1)

        pltpu.sync_copy(b1, acc.at[dstv.at[gb]], add=True)
        return carry

    lax.fori_loop(0, GM // 2, piter, 0)
    plsc.subcore_barrier()

    def wb_row(b):
        pltpu.sync_copy(acc.at[pl.ds(b, K)], zsrc)
        pltpu.sync_copy(zsrc, sums_hbm.at[c, pl.ds(b, K)])

    split_copy(wb_row)
    plsc.subcore_barrier()

    fill(b0, 1.0, KM)
    fill(b1, 0.0, K)
    z1 = b1.at[pl.ds(0, K)]
    split_copy(lambda b: pltpu.sync_copy(z1, acc.at[pl.ds(b, K)]))
    plsc.subcore_barrier()

    def cissue(g):
        pltpu.async_copy(b0, acc.at[cdstv.at[g]], sem_c, add=True)

    def cwait(g):
        pltpu.make_async_copy(b0, acc.at[cdstv.at[g]], sem_c).wait()

    def cloop(g, carry):
        cissue(g)

        @pl.when(g >= 4)
        def _():
            cwait(g - 4)

        return carry

    lax.fori_loop(0, GCM, cloop, 0)
    for t in range(4):
        cwait(GCM - 4 + t)
    plsc.subcore_barrier()

    def wb_cnt(b):
        pltpu.sync_copy(acc.at[pl.ds(b, K)], z1)
        pltpu.sync_copy(z1, cnt_hbm.at[c, pl.ds(b, K)])

    split_copy(wb_cnt)


def _sc_call(m, src1, w1_, dstm, cdstm):
    mesh = plsc.VectorSubcoreMesh(core_axis_name="c", subcore_axis_name="s")
    kern = pl.kernel(
        _sc_body,
        out_type=(jax.ShapeDtypeStruct((NC, N, H), f32),
                  jax.ShapeDtypeStruct((NC, N, H), f32)),
        mesh=mesh,
        scratch_types=(
            pltpu.VMEM_SHARED((N + 8, H), f32),
            pltpu.VMEM((GM, KM), i32),
            pltpu.VMEM((GCM, KM), i32),
            pltpu.VMEM((KM, H), f32),
            pltpu.VMEM((KM, H), f32),
            pltpu.VMEM((KM,), i32),
            pltpu.VMEM((KM,), i32),
            pltpu.VMEM((KM,), f32),
            pltpu.VMEM((KM,), f32),
            pltpu.SemaphoreType.DMA,
            pltpu.SemaphoreType.DMA,
            pltpu.SemaphoreType.DMA,
            pltpu.SemaphoreType.DMA,
        ),
    )
    return kern(m, src1, w1_, dstm, cdstm)


def _upd_body(x_ref, a0_ref, a1_ref, c0_ref, c1_ref, wa_ref, w0_ref, w1_ref,
              b_ref, o_ref):
    cnt = c0_ref[...][:, 0:1] + c1_ref[...][:, 0:1]
    inv = 1.0 / jnp.maximum(cnt, 1.0)
    y = jnp.dot(x_ref[...], wa_ref[...], preferred_element_type=f32)
    y += jnp.dot(a0_ref[...] * inv, w0_ref[...], preferred_element_type=f32)
    y += jnp.dot(a1_ref[...] * inv, w1_ref[...], preferred_element_type=f32)
    o_ref[...] = jnp.maximum(y + b_ref[...], 0.0)


def _upd_call(x, a0, a1, c0, c1, wa, w0, w1, b2):
    rb = 1000
    grid = (N // rb,)
    return pl.pallas_call(
        _upd_body,
        grid=grid,
        in_specs=[
            pl.BlockSpec((rb, D), lambda i: (i, 0)),
            pl.BlockSpec((rb, H), lambda i: (i, 0)),
            pl.BlockSpec((rb, H), lambda i: (i, 0)),
            pl.BlockSpec((rb, H), lambda i: (i, 0)),
            pl.BlockSpec((rb, H), lambda i: (i, 0)),
            pl.BlockSpec((D, D), lambda i: (0, 0)),
            pl.BlockSpec((H, D), lambda i: (0, 0)),
            pl.BlockSpec((H, D), lambda i: (0, 0)),
            pl.BlockSpec((1, D), lambda i: (0, 0)),
        ],
        out_specs=pl.BlockSpec((rb, D), lambda i: (i, 0)),
        out_shape=jax.ShapeDtypeStruct((N, D), f32),
    )(x, a0, a1, c0, c1, wa, w0, w1, b2)


@jax.jit
def _run(node_reps, edges, edge_weights, prep_gamma, prep_beta, prep_mean,
         prep_var, prep_W, prep_b, upd_gamma, upd_beta, upd_mean, upd_var,
         upd_W, upd_b):
    ap = prep_gamma / jnp.sqrt(prep_var + 1e-3)
    cp = prep_beta - prep_mean * ap
    w1 = prep_W * ap[:, None]
    b1 = (cp @ prep_W + prep_b)[None, :]

    au = upd_gamma / jnp.sqrt(upd_var + 1e-3)
    cu = upd_beta - upd_mean * au
    w2 = upd_W * au[:, None]
    b2 = (cu @ upd_W + upd_b)[None, :]
    wa, w0, w1u = w2[:D], w2[D:D + H], w2[D + H:]

    m = _prep_call(node_reps, w1, b1)

    npad = ETP - ET
    s2 = jnp.pad(edges[1].reshape(NS, ET), ((0, 0), (0, npad)))
    src1 = jnp.concatenate([s2, s2 + N]).reshape(-1)
    d2 = jnp.pad(edges[0].reshape(NS, ET), ((0, 0), (0, npad)),
                 constant_values=N)
    dm = d2.reshape(NS, GM, KM)
    w2_ = jnp.pad(edge_weights.reshape(NS, ET), ((0, 0), (0, npad)))
    w1_ = w2_.reshape(-1)
    c2 = jnp.pad(edges[0].reshape(NC * NS, ETC),
                 ((0, 0), (0, ETCP - ETC)), constant_values=N)
    cdstm = c2.reshape(NC * NS, GCM, KM)
    sums, cnt = _sc_call(m, src1, w1_, dm, cdstm)

    return _upd_call(node_reps, sums[0], sums[1], cnt[0], cnt[1],
                     wa, w0, w1u, b2)


def kernel(node_reps, edges, edge_weights, prep_gamma, prep_beta, prep_mean,
           prep_var, prep_W, prep_b, upd_gamma, upd_beta, upd_mean, upd_var,
           upd_W, upd_b):
    return _run(node_reps, edges, edge_weights, prep_gamma, prep_beta,
                prep_mean, prep_var, prep_W, prep_b, upd_gamma, upd_beta,
                upd_mean, upd_var, upd_W, upd_b)
